# Initial kernel scaffold; baseline (speedup 1.0000x reference)
import jax
import jax.numpy as jnp
from jax.experimental import pallas as pl


def kernel(x, edge_index, edge_attr, u, batch):
    raise NotImplementedError("write your pallas kernel here")


# speedup vs baseline: 5.5248x; 5.5248x over previous
import functools

import jax
import jax.numpy as jnp
from jax import lax
from jax.experimental import pallas as pl
from jax.experimental.pallas import tpu as pltpu
from jax.experimental.pallas import tpu_sc as plsc

_N_NODES = 10000
_N_EDGES = 320000
_DE = 16
_G = 125
_GPC = 16
_CHUNK = _G * _GPC
_NW = 32
_EPT = _N_EDGES // _NW
_CPT = _EPT // _CHUNK
_RPT = _N_NODES // 16


def _sc_scatter(edge_attr, col2d, zeros):
    mesh = plsc.VectorSubcoreMesh(core_axis_name="c", subcore_axis_name="s")

    @functools.partial(
        pl.kernel,
        out_type=[
            jax.ShapeDtypeStruct((2, _N_NODES, _DE), jnp.float32),
            jax.ShapeDtypeStruct((_N_EDGES,), jnp.float32),
        ],
        mesh=mesh,
        compiler_params=pltpu.CompilerParams(use_tc_tiling_on_sc=False,
                                             needs_layout_passes=False),
        scratch_types=[
            pltpu.VMEM_SHARED((_N_NODES, _DE), jnp.float32),
            pltpu.VMEM((_GPC, _G), jnp.int32),
            pltpu.VMEM((_CHUNK, _DE), jnp.float32),
            pltpu.VMEM((_CHUNK,), jnp.float32),
            pltpu.SemaphoreType.DMA,
        ],
    )
    def k(edge_hbm, col_hbm, zeros_hbm, part_hbm, out2_hbm, acc, idx_v, rows_v,
          out2_v, sem):
        i32 = jnp.int32
        c = lax.axis_index("c").astype(i32)
        s = lax.axis_index("s").astype(i32)
        wid = s * i32(2) + c

        pltpu.sync_copy(zeros_hbm.at[pl.ds(s * i32(_RPT), _RPT)],
                        acc.at[pl.ds(s * i32(_RPT), _RPT)])
        plsc.subcore_barrier()

        lane = lax.iota(jnp.int32, 16)
        col15 = jnp.full((16,), _DE - 1, jnp.int32)

        def chunk_body(i, carry):
            e0 = wid * i32(_EPT) + i * i32(_CHUNK)
            g0 = wid * i32(_EPT // _G) + i * i32(_GPC)
            pltpu.sync_copy(col_hbm.at[pl.ds(g0, _GPC)], idx_v)
            pltpu.sync_copy(edge_hbm.at[pl.ds(e0, _CHUNK)], rows_v)
            descs = [
                pltpu.async_copy(rows_v.at[pl.ds(j * _G, _G)],
                                 acc.at[idx_v.at[i32(j)]], sem, add=True)
                for j in range(_GPC)
            ]

            def ext_body(kk, carry2):
                vals = plsc.load_gather(rows_v, [kk * i32(16) + lane, col15])
                out2_v[pl.ds(kk * i32(16), 16)] = vals
                return carry2

            lax.fori_loop(i32(0), i32(_CHUNK // 16), ext_body, i32(0))
            pltpu.sync_copy(out2_v, out2_hbm.at[pl.ds(e0, _CHUNK)])
            for d in descs:
                d.wait()
            return carry

        lax.fori_loop(i32(0), i32(_CPT), chunk_body, i32(0))

        plsc.subcore_barrier()
        pltpu.sync_copy(acc.at[pl.ds(s * i32(_RPT), _RPT)],
                        part_hbm.at[c].at[pl.ds(s * i32(_RPT), _RPT)])

    return k(edge_attr, col2d, zeros)


def _combine(p_ref, o_ref):
    o_ref[...] = p_ref[0] + p_ref[1]


def kernel(x, edge_index, edge_attr, u, batch):
    col2d = edge_index[1].astype(jnp.int32).reshape(_N_EDGES // _G, _G)
    ea = edge_attr.astype(jnp.float32)
    zeros = jnp.zeros((_N_NODES, _DE), jnp.float32)
    part, out2 = _sc_scatter(ea, col2d, zeros)
    p = part.reshape(2, (_N_NODES * _DE) // 128, 128)
    summed16 = pl.pallas_call(
        _combine,
        out_shape=jax.ShapeDtypeStruct(((_N_NODES * _DE) // 128, 128),
                                       jnp.float32),
    )(p)
    summed = summed16.reshape(_N_NODES, _DE)[:, : _DE - 1]
    return (summed, out2)


# speedup vs baseline: 7.7981x; 1.4115x over previous
import functools

import jax
import jax.numpy as jnp
from jax import lax
from jax.experimental import pallas as pl
from jax.experimental.pallas import tpu as pltpu
from jax.experimental.pallas import tpu_sc as plsc

_N_NODES = 10000
_N_EDGES = 320000
_DE = 16
_G = 125
_GPC = 16
_CHUNK = _G * _GPC
_NW = 32
_EPT = _N_EDGES // _NW
_CPT = _EPT // _CHUNK
_RPT = _N_NODES // 16


def _sc_scatter(ea_t, col2d, zeros):
    mesh = plsc.VectorSubcoreMesh(core_axis_name="c", subcore_axis_name="s")

    @functools.partial(
        pl.kernel,
        out_type=[
            jax.ShapeDtypeStruct((2, _N_NODES, _DE), jnp.float32),
            jax.ShapeDtypeStruct((_N_EDGES,), jnp.float32),
        ],
        mesh=mesh,
        compiler_params=pltpu.CompilerParams(use_tc_tiling_on_sc=False,
                                             needs_layout_passes=False),
        scratch_types=[
            pltpu.VMEM_SHARED((_N_NODES, _DE), jnp.float32),
            pltpu.VMEM((_GPC, _G), jnp.int32),
            pltpu.VMEM((_DE, _CHUNK), jnp.float32),
            pltpu.VMEM((_CHUNK, _DE), jnp.float32),
            pltpu.SemaphoreType.DMA,
        ],
    )
    def k(ea_hbm, col_hbm, zeros_hbm, part_hbm, out2_hbm, acc, idx_v, cols_v,
          rows_v, sem):
        i32 = jnp.int32
        c = lax.axis_index("c").astype(i32)
        s = lax.axis_index("s").astype(i32)
        wid = s * i32(2) + c

        pltpu.sync_copy(zeros_hbm.at[pl.ds(s * i32(_RPT), _RPT)],
                        acc.at[pl.ds(s * i32(_RPT), _RPT)])
        plsc.subcore_barrier()

        lane = lax.iota(jnp.int32, 16)
        rots = [jnp.bitwise_and(lane + i32(kk), i32(15)) for kk in range(16)]

        def chunk_body(i, carry):
            e0 = wid * i32(_EPT) + i * i32(_CHUNK)
            g0 = wid * i32(_EPT // _G) + i * i32(_GPC)
            pltpu.sync_copy(col_hbm.at[pl.ds(g0, _GPC)], idx_v)
            pltpu.sync_copy(ea_hbm.at[:, pl.ds(e0, _CHUNK)], cols_v)
            descs = [pltpu.async_copy(cols_v.at[i32(_DE - 1)],
                                      out2_hbm.at[pl.ds(e0, _CHUNK)], sem)]

            def tr_body(v, carry2):
                base = v * i32(16)
                for kk in range(16):
                    eidx = base + rots[kk]
                    x = plsc.load_gather(cols_v, [lane, eidx])
                    plsc.store_scatter(rows_v, [eidx, lane], x)
                return carry2

            lax.fori_loop(i32(0), i32(_CHUNK // 16), tr_body, i32(0))

            descs += [
                pltpu.async_copy(rows_v.at[pl.ds(j * _G, _G)],
                                 acc.at[idx_v.at[i32(j)]], sem, add=True)
                for j in range(_GPC)
            ]
            for d in descs:
                d.wait()
            return carry

        lax.fori_loop(i32(0), i32(_CPT), chunk_body, i32(0))

        plsc.subcore_barrier()
        pltpu.sync_copy(acc.at[pl.ds(s * i32(_RPT), _RPT)],
                        part_hbm.at[c].at[pl.ds(s * i32(_RPT), _RPT)])

    return k(ea_t, col2d, zeros)


def _combine(p_ref, o_ref):
    o_ref[...] = p_ref[0] + p_ref[1]


def kernel(x, edge_index, edge_attr, u, batch):
    col2d = edge_index[1].astype(jnp.int32).reshape(_N_EDGES // _G, _G)
    ea_t = edge_attr.astype(jnp.float32).T
    zeros = jnp.zeros((_N_NODES, _DE), jnp.float32)
    part, out2 = _sc_scatter(ea_t, col2d, zeros)
    p = part.reshape(2, (_N_NODES * _DE) // 128, 128)
    summed16 = pl.pallas_call(
        _combine,
        out_shape=jax.ShapeDtypeStruct(((_N_NODES * _DE) // 128, 128),
                                       jnp.float32),
    )(p)
    summed = summed16.reshape(_N_NODES, _DE)[:, : _DE - 1]
    return (summed, out2)


# speedup vs baseline: 10.0638x; 1.2905x over previous
import functools

import jax
import jax.numpy as jnp
from jax import lax
from jax.experimental import pallas as pl
from jax.experimental.pallas import tpu as pltpu
from jax.experimental.pallas import tpu_sc as plsc

_N_NODES = 10000
_N_EDGES = 320000
_DE = 16
_G = 125
_GPC = 16
_CHUNK = _G * _GPC
_NW = 32
_EPT = _N_EDGES // _NW
_CPT = _EPT // _CHUNK
_RPT = _N_NODES // 16


def _sc_scatter(ea_t, col2d, zeros):
    mesh = plsc.VectorSubcoreMesh(core_axis_name="c", subcore_axis_name="s")

    @functools.partial(
        pl.kernel,
        out_type=[
            jax.ShapeDtypeStruct((2, _N_NODES, _DE), jnp.float32),
            jax.ShapeDtypeStruct((_N_EDGES,), jnp.float32),
        ],
        mesh=mesh,
        compiler_params=pltpu.CompilerParams(use_tc_tiling_on_sc=False,
                                             needs_layout_passes=False),
        scratch_types=[
            pltpu.VMEM_SHARED((_N_NODES, _DE), jnp.float32),
            pltpu.VMEM((_GPC, _G), jnp.int32),
            pltpu.VMEM((_DE, _CHUNK), jnp.float32),
            pltpu.VMEM((_CHUNK, _DE), jnp.float32),
            pltpu.SemaphoreType.DMA,
        ],
    )
    def k(ea_hbm, col_hbm, zeros_hbm, part_hbm, out2_hbm, acc, idx_v, cols_v,
          rows_v, sem):
        i32 = jnp.int32
        c = lax.axis_index("c").astype(i32)
        s = lax.axis_index("s").astype(i32)
        wid = s * i32(2) + c

        pltpu.sync_copy(zeros_hbm.at[pl.ds(s * i32(_RPT), _RPT)],
                        acc.at[pl.ds(s * i32(_RPT), _RPT)])
        plsc.subcore_barrier()

        lane = lax.iota(jnp.int32, 16)
        rots = [jnp.bitwise_and(lane + i32(kk), i32(15)) for kk in range(16)]

        def chunk_body(i, carry):
            e0 = wid * i32(_EPT) + i * i32(_CHUNK)
            g0 = wid * i32(_EPT // _G) + i * i32(_GPC)
            pltpu.sync_copy(col_hbm.at[pl.ds(g0, _GPC)], idx_v)
            pltpu.sync_copy(ea_hbm.at[:, pl.ds(e0, _CHUNK)], cols_v)
            descs = [pltpu.async_copy(cols_v.at[i32(_DE - 1)],
                                      out2_hbm.at[pl.ds(e0, _CHUNK)], sem)]

            def tr_body(v, carry2):
                base = v * i32(16)
                eidxs = [base + rots[kk] for kk in range(16)]
                xs = [plsc.load_gather(cols_v, [lane, eidxs[kk]])
                      for kk in range(16)]
                for kk in range(16):
                    plsc.store_scatter(rows_v, [eidxs[kk], lane], xs[kk])
                return carry2

            lax.fori_loop(i32(0), i32(_CHUNK // 16), tr_body, i32(0))

            descs += [
                pltpu.async_copy(rows_v.at[pl.ds(j * _G, _G)],
                                 acc.at[idx_v.at[i32(j)]], sem, add=True)
                for j in range(_GPC)
            ]
            for d in descs:
                d.wait()
            return carry

        lax.fori_loop(i32(0), i32(_CPT), chunk_body, i32(0))

        plsc.subcore_barrier()
        pltpu.sync_copy(acc.at[pl.ds(s * i32(_RPT), _RPT)],
                        part_hbm.at[c].at[pl.ds(s * i32(_RPT), _RPT)])

    return k(ea_t, col2d, zeros)


def _combine(p_ref, o_ref):
    o_ref[...] = p_ref[0] + p_ref[1]


def kernel(x, edge_index, edge_attr, u, batch):
    col2d = edge_index[1].astype(jnp.int32).reshape(_N_EDGES // _G, _G)
    ea_t = edge_attr.astype(jnp.float32).T
    zeros = jnp.zeros((_N_NODES, _DE), jnp.float32)
    part, out2 = _sc_scatter(ea_t, col2d, zeros)
    p = part.reshape(2, (_N_NODES * _DE) // 128, 128)
    summed16 = pl.pallas_call(
        _combine,
        out_shape=jax.ShapeDtypeStruct(((_N_NODES * _DE) // 128, 128),
                                       jnp.float32),
    )(p)
    summed = summed16.reshape(_N_NODES, _DE)[:, : _DE - 1]
    return (summed, out2)
